# baseline (speedup 1.0000x reference)
import jax
import jax.numpy as jnp
from jax.experimental import pallas as pl

E = 8
TOP_K = 2
H = 1024
FFN = 512
SFFN = 2048
TM = 256


def _moe_body(x_ref, rw_ref, gu_ref, dn_ref, out_ref):
    x = x_ref[...]

    logits = jax.lax.dot_general(x, rw_ref[...], (((1,), (1,)), ((), ())),
                                 preferred_element_type=jnp.float32)
    col = jax.lax.broadcasted_iota(jnp.int32, (TM, E), 1)
    m1 = jnp.max(logits, axis=1, keepdims=True)
    a1 = jnp.min(jnp.where(logits == m1, col, E), axis=1, keepdims=True)
    logits2 = jnp.where(col == a1, -jnp.inf, logits)
    m2 = jnp.max(logits2, axis=1, keepdims=True)
    a2 = jnp.min(jnp.where(logits2 == m2, col, E), axis=1, keepdims=True)
    keep = (col == a1) | (col == a2)
    scores = jnp.where(keep, jax.nn.sigmoid(logits), 0.0)

    acc = jnp.zeros((TM, H), jnp.float32)
    for e in range(E):
        xs = (x * scores[:, e:e + 1]).astype(jnp.bfloat16)
        gu = jnp.dot(xs, gu_ref[e], preferred_element_type=jnp.float32)
        g = gu[:, :FFN]
        u = gu[:, FFN:]
        h = (u * (g * jax.nn.sigmoid(g))).astype(jnp.bfloat16)
        acc = acc + jnp.dot(h, dn_ref[e], preferred_element_type=jnp.float32)

    out_ref[...] = acc


def _shared_body(x_ref, shg_ref, shu_ref, shd_ref, part_ref, out_ref):
    x = x_ref[...].astype(jnp.bfloat16)
    gsh = jax.lax.dot_general(x, shg_ref[...], (((1,), (1,)), ((), ())),
                              preferred_element_type=jnp.float32)
    ush = jax.lax.dot_general(x, shu_ref[...], (((1,), (1,)), ((), ())),
                              preferred_element_type=jnp.float32)
    hsh = (ush * (gsh * jax.nn.sigmoid(gsh))).astype(jnp.bfloat16)
    out_ref[...] = part_ref[...] + jax.lax.dot_general(
        hsh, shd_ref[...], (((1,), (1,)), ((), ())),
        preferred_element_type=jnp.float32)


@jax.jit
def _moe(hidden, router_w, gate_up_proj, down_proj, sh_gate_w, sh_up_w, sh_down_w):
    T = hidden.shape[0]
    part = pl.pallas_call(
        _moe_body,
        grid=(T // TM,),
        in_specs=[
            pl.BlockSpec((TM, H), lambda t: (t, 0)),
            pl.BlockSpec((E, H), lambda t: (0, 0)),
            pl.BlockSpec((E, H, 2 * FFN), lambda t: (0, 0, 0)),
            pl.BlockSpec((E, FFN, H), lambda t: (0, 0, 0)),
        ],
        out_specs=pl.BlockSpec((TM, H), lambda t: (t, 0)),
        out_shape=jax.ShapeDtypeStruct((T, H), jnp.float32),
    )(hidden, router_w, gate_up_proj.astype(jnp.bfloat16),
      down_proj.astype(jnp.bfloat16))
    out = pl.pallas_call(
        _shared_body,
        grid=(T // TM,),
        in_specs=[
            pl.BlockSpec((TM, H), lambda t: (t, 0)),
            pl.BlockSpec((SFFN, H), lambda t: (0, 0)),
            pl.BlockSpec((SFFN, H), lambda t: (0, 0)),
            pl.BlockSpec((H, SFFN), lambda t: (0, 0)),
            pl.BlockSpec((TM, H), lambda t: (t, 0)),
        ],
        out_specs=pl.BlockSpec((TM, H), lambda t: (t, 0)),
        out_shape=jax.ShapeDtypeStruct((T, H), jnp.float32),
    )(hidden, sh_gate_w.astype(jnp.bfloat16), sh_up_w.astype(jnp.bfloat16),
      sh_down_w.astype(jnp.bfloat16), part)
    return out


def kernel(hidden_states, router_w, gate_up_proj, down_proj, sh_gate_w, sh_up_w, sh_down_w):
    B, S, Hd = hidden_states.shape
    hidden = hidden_states.reshape(-1, Hd)
    out = _moe(hidden, router_w, gate_up_proj, down_proj, sh_gate_w, sh_up_w, sh_down_w)
    return out.reshape(B, S, Hd)


# speedup vs baseline: 1.2367x; 1.2367x over previous
import jax
import jax.numpy as jnp
from jax.experimental import pallas as pl

E = 8
TOP_K = 2
H = 1024
FFN = 512
SFFN = 2048
TM = 256


def _moe_body(x_ref, rw_ref, gu_ref, dn_ref, out_ref):
    x = x_ref[...]

    logits = jax.lax.dot_general(x, rw_ref[...], (((1,), (1,)), ((), ())),
                                 preferred_element_type=jnp.float32)
    col = jax.lax.broadcasted_iota(jnp.int32, (TM, E), 1)
    m1 = jnp.max(logits, axis=1, keepdims=True)
    a1 = jnp.min(jnp.where(logits == m1, col, E), axis=1, keepdims=True)
    logits2 = jnp.where(col == a1, -jnp.inf, logits)
    m2 = jnp.max(logits2, axis=1, keepdims=True)
    a2 = jnp.min(jnp.where(logits2 == m2, col, E), axis=1, keepdims=True)
    keep = (col == a1) | (col == a2)
    scores = jnp.where(keep, jax.nn.sigmoid(logits), 0.0)

    acc = jnp.zeros((TM, H), jnp.float32)
    for e in range(E):
        xs = x * scores[:, e:e + 1]
        gu = jnp.dot(xs, gu_ref[e], preferred_element_type=jnp.float32)
        g = gu[:, :FFN]
        u = gu[:, FFN:]
        h = u * (g * jax.nn.sigmoid(g))
        acc = acc + jnp.dot(h, dn_ref[e], preferred_element_type=jnp.float32)

    out_ref[...] = acc


def _shared_body(x_ref, shg_ref, shu_ref, shd_ref, part_ref, out_ref):
    x = x_ref[...]
    gsh = jax.lax.dot_general(x, shg_ref[...], (((1,), (1,)), ((), ())),
                              preferred_element_type=jnp.float32)
    ush = jax.lax.dot_general(x, shu_ref[...], (((1,), (1,)), ((), ())),
                              preferred_element_type=jnp.float32)
    hsh = ush * (gsh * jax.nn.sigmoid(gsh))
    out_ref[...] = part_ref[...] + jax.lax.dot_general(
        hsh, shd_ref[...], (((1,), (1,)), ((), ())),
        preferred_element_type=jnp.float32)


@jax.jit
def _moe(hidden, router_w, gate_up_proj, down_proj, sh_gate_w, sh_up_w, sh_down_w):
    T = hidden.shape[0]
    part = pl.pallas_call(
        _moe_body,
        grid=(T // TM,),
        in_specs=[
            pl.BlockSpec((TM, H), lambda t: (t, 0)),
            pl.BlockSpec((E, H), lambda t: (0, 0)),
            pl.BlockSpec((E, H, 2 * FFN), lambda t: (0, 0, 0)),
            pl.BlockSpec((E, FFN, H), lambda t: (0, 0, 0)),
        ],
        out_specs=pl.BlockSpec((TM, H), lambda t: (t, 0)),
        out_shape=jax.ShapeDtypeStruct((T, H), jnp.float32),
    )(hidden, router_w, gate_up_proj, down_proj)
    out = pl.pallas_call(
        _shared_body,
        grid=(T // TM,),
        in_specs=[
            pl.BlockSpec((TM, H), lambda t: (t, 0)),
            pl.BlockSpec((SFFN, H), lambda t: (0, 0)),
            pl.BlockSpec((SFFN, H), lambda t: (0, 0)),
            pl.BlockSpec((H, SFFN), lambda t: (0, 0)),
            pl.BlockSpec((TM, H), lambda t: (t, 0)),
        ],
        out_specs=pl.BlockSpec((TM, H), lambda t: (t, 0)),
        out_shape=jax.ShapeDtypeStruct((T, H), jnp.float32),
    )(hidden, sh_gate_w, sh_up_w, sh_down_w, part)
    return out


def kernel(hidden_states, router_w, gate_up_proj, down_proj, sh_gate_w, sh_up_w, sh_down_w):
    B, S, Hd = hidden_states.shape
    hidden = hidden_states.reshape(-1, Hd)
    out = _moe(hidden, router_w, gate_up_proj, down_proj, sh_gate_w, sh_up_w, sh_down_w)
    return out.reshape(B, S, Hd)
